# Initial kernel scaffold; baseline (speedup 1.0000x reference)
import jax
import jax.numpy as jnp
from jax.experimental import pallas as pl


def kernel(q_u, q_b, q_sigma, delta_rule_proto, signature_proto, write_mass, ema_conf):
    raise NotImplementedError("write your pallas kernel here")


# speedup vs baseline: 2.4772x; 2.4772x over previous
import jax
import jax.numpy as jnp
from jax import lax
from jax.experimental import pallas as pl
from jax.experimental.pallas import tpu as pltpu

N_OPS = 128
N_BIND = 128
N_CELLS = N_OPS * N_BIND
SIG_DIM = 64
AUG = 72
RULE_DIM = 64
BLOCK_N = 128


def _rule_memory_block(q_u_ref, q_b_ref, q_sigma_ref, catp_ref,
                       sig_t_ref, wm_ref, ec_ref,
                       delta_out_ref, msig_out_ref, conf_out_ref,
                       w_out_ref, topw_out_ref,
                       pnt_ref, krow_ref):
    f32 = jnp.float32
    i = pl.program_id(0)

    @pl.when(i == 0)
    def _init():
        sp = sig_t_ref[...] + 1e-6
        nrm = jnp.sqrt(jnp.sum(sp * sp, axis=0, keepdims=True))
        pnt_ref[0:SIG_DIM, :] = 0.5 * (sp / jnp.maximum(nrm, 1e-12))
        pnt_ref[SIG_DIM:SIG_DIM + 1, :] = jnp.full((1, N_CELLS), 0.5, f32)
        pnt_ref[SIG_DIM + 1:, :] = jnp.zeros((AUG - SIG_DIM - 1, N_CELLS), f32)
        krow_ref[...] = lax.broadcasted_iota(
            jnp.int32, (1, N_CELLS), 1).astype(f32)

    wm = wm_ref[...]
    ec = ec_ref[...]
    usage = jnp.log1p(wm)
    usage = usage / jnp.maximum(jnp.max(usage, axis=1, keepdims=True), 1.0)
    confp = ec / jnp.maximum(jnp.max(ec, axis=1, keepdims=True), 1e-6)
    prior = jnp.where(wm > 0.0, 0.5 * usage + 0.5 * confp, -1e9)

    qs = q_sigma_ref[...]
    qn = qs / jnp.maximum(
        jnp.sqrt(jnp.sum(qs * qs, axis=1, keepdims=True)), 1e-12)
    qn_aug = jnp.concatenate(
        [qn, jnp.ones((BLOCK_N, 1), f32),
         jnp.zeros((BLOCK_N, AUG - SIG_DIM - 1), f32)], axis=1)
    sig = jnp.dot(qn_aug, pnt_ref[...], preferred_element_type=f32)

    joint3 = q_u_ref[...][:, :, None] * q_b_ref[...][:, None, :]
    jlog = jnp.log(jnp.maximum(joint3, 1e-6)).reshape(BLOCK_N, N_CELLS)

    masked = (jlog + prior) + sig
    mx = jnp.max(masked, axis=1, keepdims=True)
    e = jnp.exp(masked - mx)

    s = jnp.sum(e, axis=1, keepdims=True)
    krow = krow_ref[...]
    cand = jnp.where(masked == mx, krow, float(N_CELLS))
    idx = jnp.min(cand, axis=1, keepdims=True)

    inv = 1.0 / jnp.maximum(s, 1e-30)
    w_out_ref[...] = e * inv
    raw = jnp.dot(e, catp_ref[...], preferred_element_type=f32)
    delta_out_ref[...] = raw[:, 0:RULE_DIM] * inv
    msig_out_ref[...] = raw[:, RULE_DIM:RULE_DIM + SIG_DIM] * inv

    topw = inv
    topw_out_ref[...] = topw
    top_cs = jnp.sum(
        jnp.where(krow == idx, confp * sig, 0.0),
        axis=1, keepdims=True)
    conf_out_ref[...] = jnp.clip(topw * top_cs, 0.0, 1.0)


@jax.jit
def kernel(q_u, q_b, q_sigma, delta_rule_proto, signature_proto,
           write_mass, ema_conf):
    batch = q_u.shape[0]
    f32 = jnp.float32
    dflat = delta_rule_proto.reshape(N_CELLS, RULE_DIM)
    sflat = signature_proto.reshape(N_CELLS, SIG_DIM)
    catp = jnp.concatenate([dflat, sflat], axis=1)
    sig_t = sflat.T
    wmf = write_mass.reshape(1, N_CELLS)
    ecf = ema_conf.reshape(1, N_CELLS)

    grid = (batch // BLOCK_N,)
    row = lambda i: (i, 0)
    full = lambda i: (0, 0)
    in_specs = [
        pl.BlockSpec((BLOCK_N, N_OPS), row),
        pl.BlockSpec((BLOCK_N, N_BIND), row),
        pl.BlockSpec((BLOCK_N, SIG_DIM), row),
        pl.BlockSpec((N_CELLS, RULE_DIM + SIG_DIM), full),
        pl.BlockSpec((SIG_DIM, N_CELLS), full),
        pl.BlockSpec((1, N_CELLS), full),
        pl.BlockSpec((1, N_CELLS), full),
    ]
    out_specs = (
        pl.BlockSpec((BLOCK_N, RULE_DIM), row),
        pl.BlockSpec((BLOCK_N, SIG_DIM), row),
        pl.BlockSpec((BLOCK_N, 1), row),
        pl.BlockSpec((BLOCK_N, N_CELLS), row),
        pl.BlockSpec((BLOCK_N, 1), row),
    )
    out_shape = (
        jax.ShapeDtypeStruct((batch, RULE_DIM), f32),
        jax.ShapeDtypeStruct((batch, SIG_DIM), f32),
        jax.ShapeDtypeStruct((batch, 1), f32),
        jax.ShapeDtypeStruct((batch, N_CELLS), f32),
        jax.ShapeDtypeStruct((batch, 1), f32),
    )
    delta, msig, conf, w, topw = pl.pallas_call(
        _rule_memory_block,
        grid=grid,
        in_specs=in_specs,
        out_specs=out_specs,
        out_shape=out_shape,
        scratch_shapes=[
            pltpu.VMEM((AUG, N_CELLS), f32),
            pltpu.VMEM((1, N_CELLS), f32),
        ],
    )(q_u, q_b, q_sigma, catp, sig_t, wmf, ecf)
    return delta, msig, conf, w.reshape(batch, N_OPS, N_BIND), topw
